# baseline (device time: 392060 ns/iter reference)
import jax
import jax.numpy as jnp
from jax import lax
from jax.experimental import pallas as pl
from jax.experimental.pallas import tpu as pltpu

N_DEV = 16


def kernel(x, w_mat):
    m_per, k = x.shape
    _, n_per = w_mat.shape

    def body(x_ref, w_ref, out_ref, comm_ref, wb_ref, amax_ref,
             ring_send_sems, ring_recv_sems, a_send_sems, a_recv_sems):
        my = lax.axis_index("i")
        left = lax.rem(my - 1 + N_DEV, N_DEV)
        right = lax.rem(my + 1, N_DEV)

        barrier_sem = pltpu.get_barrier_semaphore()
        for nbr in (left, right):
            pl.semaphore_signal(
                barrier_sem, inc=1,
                device_id=(nbr,), device_id_type=pl.DeviceIdType.MESH,
            )
        pl.semaphore_wait(barrier_sem, 2)

        wb_ref[...] = w_ref[...].astype(jnp.bfloat16)
        comm_ref[0, :, :] = x_ref[...].astype(jnp.bfloat16)

        def gemm(chunk, origin):
            y = lax.dot_general(
                chunk, wb_ref[...], (((1,), (0,)), ((), ())),
                preferred_element_type=jnp.float32,
            )
            y = jnp.maximum(y, 0.0)
            out_ref[pl.ds(origin * m_per, m_per), :] = y
            return jnp.max(y)

        local_amax = gemm(comm_ref[0, :, :], my)

        for h in range(N_DEV - 1):
            s, r = h % 2, (h + 1) % 2
            rdma = pltpu.make_async_remote_copy(
                src_ref=comm_ref.at[s],
                dst_ref=comm_ref.at[r],
                send_sem=ring_send_sems.at[s],
                recv_sem=ring_recv_sems.at[r],
                device_id=(right,),
                device_id_type=pl.DeviceIdType.MESH,
            )
            rdma.start()
            rdma.wait()
            origin = lax.rem(my - (h + 1) + N_DEV, N_DEV)
            local_amax = jnp.maximum(local_amax, gemm(comm_ref[r, :, :], origin))

        amax_ref[15, :, :] = jnp.full((8, 128), local_amax, jnp.float32)
        sends = []
        for d in range(1, N_DEV):
            tgt = lax.rem(my + d, N_DEV)
            c = pltpu.make_async_remote_copy(
                src_ref=amax_ref.at[15],
                dst_ref=amax_ref.at[d - 1],
                send_sem=a_send_sems.at[d - 1],
                recv_sem=a_recv_sems.at[d - 1],
                device_id=(tgt,),
                device_id_type=pl.DeviceIdType.MESH,
            )
            c.start()
            sends.append(c)
        g = local_amax
        for d, c in zip(range(1, N_DEV), sends):
            c.wait_send()
            c.wait_recv()
            g = jnp.maximum(g, amax_ref[d - 1, 0, 0])

        scale = g / 127.0
        q = jnp.clip(jnp.round(out_ref[...] / scale), -127.0, 127.0)
        out_ref[...] = q * scale

    return pl.pallas_call(
        body,
        out_shape=jax.ShapeDtypeStruct((N_DEV * m_per, n_per), jnp.float32),
        in_specs=[
            pl.BlockSpec(memory_space=pltpu.VMEM),
            pl.BlockSpec(memory_space=pltpu.VMEM),
        ],
        out_specs=pl.BlockSpec(memory_space=pltpu.VMEM),
        scratch_shapes=[
            pltpu.VMEM((2, m_per, k), jnp.bfloat16),
            pltpu.VMEM((k, n_per), jnp.bfloat16),
            pltpu.VMEM((N_DEV, 8, 128), jnp.float32),
            pltpu.SemaphoreType.DMA((2,)),
            pltpu.SemaphoreType.DMA((2,)),
            pltpu.SemaphoreType.DMA((N_DEV - 1,)),
            pltpu.SemaphoreType.DMA((N_DEV - 1,)),
        ],
        compiler_params=pltpu.CompilerParams(collective_id=0),
    )(x, w_mat)


# device time: 231389 ns/iter; 1.6944x vs baseline; 1.6944x over previous
import jax
import jax.numpy as jnp
from jax import lax
from jax.experimental import pallas as pl
from jax.experimental.pallas import tpu as pltpu

N_DEV = 16


def kernel(x, w_mat):
    m_per, k = x.shape
    _, n_per = w_mat.shape
    half = m_per // 2

    def body(x_ref, w_ref, out_ref, cw_ref, ccw_ref, wb_ref, amax_ref,
             cw_send_sems, cw_recv_sems, ccw_send_sems, ccw_recv_sems,
             a_send_sems, a_recv_sems):
        my = lax.axis_index("i")
        left = lax.rem(my - 1 + N_DEV, N_DEV)
        right = lax.rem(my + 1, N_DEV)

        barrier_sem = pltpu.get_barrier_semaphore()
        for nbr in (left, right):
            pl.semaphore_signal(
                barrier_sem, inc=1,
                device_id=(nbr,), device_id_type=pl.DeviceIdType.MESH,
            )
        pl.semaphore_wait(barrier_sem, 2)

        wb_ref[...] = w_ref[...].astype(jnp.bfloat16)
        xb = x_ref[...].astype(jnp.bfloat16)
        cw_ref[0, :, :] = xb[:half, :]
        ccw_ref[0, :, :] = xb[half:, :]

        def make(h):
            s, r = h % 2, (h + 1) % 2
            cw = pltpu.make_async_remote_copy(
                src_ref=cw_ref.at[s], dst_ref=cw_ref.at[r],
                send_sem=cw_send_sems.at[s], recv_sem=cw_recv_sems.at[r],
                device_id=(right,), device_id_type=pl.DeviceIdType.MESH,
            )
            ccw = pltpu.make_async_remote_copy(
                src_ref=ccw_ref.at[s], dst_ref=ccw_ref.at[r],
                send_sem=ccw_send_sems.at[s], recv_sem=ccw_recv_sems.at[r],
                device_id=(left,), device_id_type=pl.DeviceIdType.MESH,
            )
            return cw, ccw

        hops = [make(h) for h in range(N_DEV - 1)]
        hops[0][0].start()
        hops[0][1].start()

        def gemm(chunk, row0, rows):
            y = lax.dot_general(
                chunk, wb_ref[...], (((1,), (0,)), ((), ())),
                preferred_element_type=jnp.float32,
            )
            y = jnp.maximum(y, 0.0)
            out_ref[pl.ds(row0, rows), :] = y
            return jnp.max(y)

        local_amax = gemm(xb, my * m_per, m_per)

        for h in range(N_DEV - 1):
            cw, ccw = hops[h]
            cw.wait_recv()
            ccw.wait_recv()
            if h + 1 < N_DEV - 1:
                if h >= 1:
                    hops[h - 1][0].wait_send()
                    hops[h - 1][1].wait_send()
                hops[h + 1][0].start()
                hops[h + 1][1].start()
            r = (h + 1) % 2
            ocw = lax.rem(my - (h + 1) + N_DEV, N_DEV)
            occw = lax.rem(my + (h + 1), N_DEV)
            a1 = gemm(cw_ref[r, :, :], ocw * m_per, half)
            a2 = gemm(ccw_ref[r, :, :], occw * m_per + half, half)
            local_amax = jnp.maximum(local_amax, jnp.maximum(a1, a2))

        for h in (N_DEV - 3, N_DEV - 2):
            hops[h][0].wait_send()
            hops[h][1].wait_send()

        amax_ref[N_DEV - 1, :, :] = jnp.full((8, 128), local_amax, jnp.float32)
        sends = []
        for d in range(1, N_DEV):
            tgt = lax.rem(my + d, N_DEV)
            c = pltpu.make_async_remote_copy(
                src_ref=amax_ref.at[N_DEV - 1],
                dst_ref=amax_ref.at[d - 1],
                send_sem=a_send_sems.at[d - 1],
                recv_sem=a_recv_sems.at[d - 1],
                device_id=(tgt,),
                device_id_type=pl.DeviceIdType.MESH,
            )
            c.start()
            sends.append(c)
        g = local_amax
        for d, c in zip(range(1, N_DEV), sends):
            c.wait_send()
            c.wait_recv()
            g = jnp.maximum(g, amax_ref[d - 1, 0, 0])

        scale = g / 127.0
        q = jnp.clip(jnp.round(out_ref[...] / scale), -127.0, 127.0)
        out_ref[...] = q * scale

    return pl.pallas_call(
        body,
        out_shape=jax.ShapeDtypeStruct((N_DEV * m_per, n_per), jnp.float32),
        in_specs=[
            pl.BlockSpec(memory_space=pltpu.VMEM),
            pl.BlockSpec(memory_space=pltpu.VMEM),
        ],
        out_specs=pl.BlockSpec(memory_space=pltpu.VMEM),
        scratch_shapes=[
            pltpu.VMEM((2, half, k), jnp.bfloat16),
            pltpu.VMEM((2, half, k), jnp.bfloat16),
            pltpu.VMEM((k, n_per), jnp.bfloat16),
            pltpu.VMEM((N_DEV, 8, 128), jnp.float32),
            pltpu.SemaphoreType.DMA((2,)),
            pltpu.SemaphoreType.DMA((2,)),
            pltpu.SemaphoreType.DMA((2,)),
            pltpu.SemaphoreType.DMA((2,)),
            pltpu.SemaphoreType.DMA((N_DEV - 1,)),
            pltpu.SemaphoreType.DMA((N_DEV - 1,)),
        ],
        compiler_params=pltpu.CompilerParams(collective_id=0),
    )(x, w_mat)


# device time: 188078 ns/iter; 2.0846x vs baseline; 1.2303x over previous
import jax
import jax.numpy as jnp
from jax import lax
from jax.experimental import pallas as pl
from jax.experimental.pallas import tpu as pltpu

N_DEV = 16
SUB = 2


def kernel(x, w_mat):
    m_per, k = x.shape
    _, n_per = w_mat.shape
    half = m_per // 2
    subrows = half // SUB

    def body(x_ref, w_ref, out_ref, cw_ref, ccw_ref, wb_ref, amax_ref,
             cw_send_sems, cw_recv_sems, ccw_send_sems, ccw_recv_sems,
             a_send_sems, a_recv_sems):
        my = lax.axis_index("i")
        left = lax.rem(my - 1 + N_DEV, N_DEV)
        right = lax.rem(my + 1, N_DEV)

        barrier_sem = pltpu.get_barrier_semaphore()
        for nbr in (left, right):
            pl.semaphore_signal(
                barrier_sem, inc=1,
                device_id=(nbr,), device_id_type=pl.DeviceIdType.MESH,
            )
        pl.semaphore_wait(barrier_sem, 2)

        wb_ref[...] = w_ref[...].astype(jnp.bfloat16)
        xb = x_ref[...].astype(jnp.bfloat16)
        cw_ref[0] = xb[:half, :].reshape(SUB, subrows, k)
        ccw_ref[0] = xb[half:, :].reshape(SUB, subrows, k)

        def make(h, j):
            s, r = h % 2, (h + 1) % 2
            cw = pltpu.make_async_remote_copy(
                src_ref=cw_ref.at[s, j], dst_ref=cw_ref.at[r, j],
                send_sem=cw_send_sems.at[s, j], recv_sem=cw_recv_sems.at[r, j],
                device_id=(right,), device_id_type=pl.DeviceIdType.MESH,
            )
            ccw = pltpu.make_async_remote_copy(
                src_ref=ccw_ref.at[s, j], dst_ref=ccw_ref.at[r, j],
                send_sem=ccw_send_sems.at[s, j], recv_sem=ccw_recv_sems.at[r, j],
                device_id=(left,), device_id_type=pl.DeviceIdType.MESH,
            )
            return cw, ccw

        hops = [[make(h, j) for j in range(SUB)] for h in range(N_DEV - 1)]
        for j in range(SUB):
            hops[0][j][0].start()
            hops[0][j][1].start()

        def gemm(chunk, row0, rows):
            y = lax.dot_general(
                chunk, wb_ref[...], (((1,), (0,)), ((), ())),
                preferred_element_type=jnp.float32,
            )
            y = jnp.maximum(y, 0.0)
            out_ref[pl.ds(row0, rows), :] = y
            return jnp.max(y)

        local_amax = gemm(xb, my * m_per, m_per)

        for h in range(N_DEV - 1):
            for j in range(SUB):
                cw, ccw = hops[h][j]
                cw.wait_recv()
                ccw.wait_recv()
                if h + 1 < N_DEV - 1:
                    if h >= 1:
                        hops[h - 1][j][0].wait_send()
                        hops[h - 1][j][1].wait_send()
                    hops[h + 1][j][0].start()
                    hops[h + 1][j][1].start()
            r = (h + 1) % 2
            ocw = lax.rem(my - (h + 1) + N_DEV, N_DEV)
            occw = lax.rem(my + (h + 1), N_DEV)
            for j in range(SUB):
                a1 = gemm(cw_ref[r, j], ocw * m_per + j * subrows, subrows)
                a2 = gemm(ccw_ref[r, j],
                          occw * m_per + half + j * subrows, subrows)
                local_amax = jnp.maximum(local_amax, jnp.maximum(a1, a2))

        for h in (N_DEV - 3, N_DEV - 2):
            for j in range(SUB):
                hops[h][j][0].wait_send()
                hops[h][j][1].wait_send()

        amax_ref[N_DEV - 1, :, :] = jnp.full((8, 128), local_amax, jnp.float32)
        sends = []
        for d in range(1, N_DEV):
            tgt = lax.rem(my + d, N_DEV)
            c = pltpu.make_async_remote_copy(
                src_ref=amax_ref.at[N_DEV - 1],
                dst_ref=amax_ref.at[d - 1],
                send_sem=a_send_sems.at[d - 1],
                recv_sem=a_recv_sems.at[d - 1],
                device_id=(tgt,),
                device_id_type=pl.DeviceIdType.MESH,
            )
            c.start()
            sends.append(c)
        g = local_amax
        for d, c in zip(range(1, N_DEV), sends):
            c.wait_send()
            c.wait_recv()
            g = jnp.maximum(g, amax_ref[d - 1, 0, 0])

        scale = g / 127.0
        q = jnp.clip(jnp.round(out_ref[...] / scale), -127.0, 127.0)
        out_ref[...] = q * scale

    return pl.pallas_call(
        body,
        out_shape=jax.ShapeDtypeStruct((N_DEV * m_per, n_per), jnp.float32),
        in_specs=[
            pl.BlockSpec(memory_space=pltpu.VMEM),
            pl.BlockSpec(memory_space=pltpu.VMEM),
        ],
        out_specs=pl.BlockSpec(memory_space=pltpu.VMEM),
        scratch_shapes=[
            pltpu.VMEM((2, SUB, subrows, k), jnp.bfloat16),
            pltpu.VMEM((2, SUB, subrows, k), jnp.bfloat16),
            pltpu.VMEM((k, n_per), jnp.bfloat16),
            pltpu.VMEM((N_DEV, 8, 128), jnp.float32),
            pltpu.SemaphoreType.DMA((2, SUB)),
            pltpu.SemaphoreType.DMA((2, SUB)),
            pltpu.SemaphoreType.DMA((2, SUB)),
            pltpu.SemaphoreType.DMA((2, SUB)),
            pltpu.SemaphoreType.DMA((N_DEV - 1,)),
            pltpu.SemaphoreType.DMA((N_DEV - 1,)),
        ],
        compiler_params=pltpu.CompilerParams(collective_id=0),
    )(x, w_mat)
